# Initial kernel scaffold; baseline (speedup 1.0000x reference)
import jax
import jax.numpy as jnp
from jax.experimental import pallas as pl


def kernel(t, emb_weight, W, b):
    raise NotImplementedError("write your pallas kernel here")


# speedup vs baseline: 1.3202x; 1.3202x over previous
import functools

import jax
import jax.numpy as jnp
from jax import lax
from jax.experimental import pallas as pl
from jax.experimental.pallas import tpu as pltpu
from jax.experimental.pallas import tpu_sc as plsc


def _project_body(emb_ref, w_ref, b_ref, out_ref):
    out_ref[...] = lax.dot_general(
        emb_ref[...], w_ref[...],
        (((1,), (1,)), ((), ())),
        preferred_element_type=jnp.float32,
    ) + b_ref[...]


def _project_table(emb_weight, W, b):
    m = emb_weight.shape[0]
    return pl.pallas_call(
        _project_body,
        out_shape=jax.ShapeDtypeStruct((m, W.shape[0]), jnp.float32),
    )(emb_weight, W, b.reshape(1, -1))


@functools.lru_cache(maxsize=None)
def _make_gather(B, D, NC, NS, b_per_w, C):
    n_chunks = b_per_w // C
    mesh = plsc.VectorSubcoreMesh(
        core_axis_name="c", subcore_axis_name="s",
        num_cores=NC, num_subcores=NS)

    @functools.partial(
        pl.kernel,
        out_type=jax.ShapeDtypeStruct((B, D), jnp.float32),
        mesh=mesh,
        scratch_types=[
            pltpu.VMEM((b_per_w,), jnp.int32),
            pltpu.VMEM((C, D), jnp.float32),
            pltpu.SemaphoreType.DMA,
        ],
    )
    def gather_k(table_hbm, idx_hbm, out_hbm, idx_v, rows_v, sem):
        wid = lax.axis_index("s") * NC + lax.axis_index("c")
        base = wid * b_per_w
        pltpu.sync_copy(idx_hbm.at[pl.ds(base, b_per_w)], idx_v)

        def body(i, carry):
            off = i * C
            pltpu.async_copy(
                table_hbm.at[idx_v.at[pl.ds(off, C)]], rows_v, sem).wait()
            pltpu.sync_copy(rows_v, out_hbm.at[pl.ds(base + off, C)])
            return carry

        lax.fori_loop(0, n_chunks, body, 0)

    return gather_k


def kernel(t, emb_weight, W, b):
    B = t.shape[0]
    D = W.shape[0]
    P = _project_table(emb_weight, W, b)

    info = plsc.get_sparse_core_info()
    NC, NS = info.num_cores, info.num_subcores
    NW = NC * NS
    assert B % NW == 0, (B, NW)
    b_per_w = B // NW

    C = 0
    for cand in range(128, 0, -8):
        if b_per_w % cand == 0:
            C = cand
            break
    assert C > 0, b_per_w

    return _make_gather(B, D, NC, NS, b_per_w, C)(P, t)


# speedup vs baseline: 9.2110x; 6.9770x over previous
import functools

import jax
import jax.numpy as jnp
from jax import lax
from jax.experimental import pallas as pl
from jax.experimental.pallas import tpu as pltpu
from jax.experimental.pallas import tpu_sc as plsc


def _project_body(emb_ref, w_ref, b_ref, out_ref):
    out_ref[...] = lax.dot_general(
        emb_ref[...], w_ref[...],
        (((1,), (1,)), ((), ())),
        preferred_element_type=jnp.float32,
    ) + b_ref[...]


def _project_table(emb_weight, W, b):
    m = emb_weight.shape[0]
    return pl.pallas_call(
        _project_body,
        out_shape=jax.ShapeDtypeStruct((m, W.shape[0]), jnp.float32),
    )(emb_weight, W, b.reshape(1, -1))


@functools.lru_cache(maxsize=None)
def _make_gather(B, V, D, NC, NS, b_per_w, C, NBUF):
    n_chunks = b_per_w // C
    n_groups = n_chunks // NBUF
    mesh = plsc.VectorSubcoreMesh(
        core_axis_name="c", subcore_axis_name="s",
        num_cores=NC, num_subcores=NS)

    @functools.partial(
        pl.kernel,
        out_type=jax.ShapeDtypeStruct((B, D), jnp.float32),
        mesh=mesh,
        scratch_types=[
            pltpu.VMEM_SHARED((V, D), jnp.float32),
            pltpu.VMEM((b_per_w,), jnp.int32),
            pltpu.VMEM((NBUF, C, D), jnp.float32),
        ] + [pltpu.SemaphoreType.DMA] * (2 * NBUF),
    )
    def gather_k(table_hbm, idx_hbm, out_hbm, shared_tab, idx_v, rows_v,
                 *sems):
        gsem = sems[:NBUF]
        ssem = sems[NBUF:]
        sid = lax.axis_index("s")
        wid = sid * NC + lax.axis_index("c")
        base = wid * b_per_w

        @pl.when(sid == 0)
        def _():
            pltpu.sync_copy(table_hbm, shared_tab)

        pltpu.sync_copy(idx_hbm.at[pl.ds(base, b_per_w)], idx_v)
        plsc.subcore_barrier()

        def fire_gather(chunk, b):
            off = chunk * C
            pltpu.async_copy(
                shared_tab.at[idx_v.at[pl.ds(off, C)]], rows_v.at[b],
                gsem[b])

        def drain(sem, b):
            pltpu.make_async_copy(
                out_hbm.at[pl.ds(0, C)], rows_v.at[b], sem).wait()

        for b in range(NBUF):
            fire_gather(b, b)

        def group_body(g, carry):
            first = g * NBUF
            for b in range(NBUF):
                drain(gsem[b], b)
                pltpu.async_copy(
                    rows_v.at[b],
                    out_hbm.at[pl.ds(base + (first + b) * C, C)],
                    ssem[b])
            for b in range(NBUF):
                drain(ssem[b], b)

                @pl.when(g < n_groups - 1)
                def _():
                    fire_gather(first + NBUF + b, b)
            return carry

        lax.fori_loop(0, n_groups, group_body, 0)

    return gather_k


def kernel(t, emb_weight, W, b):
    B = t.shape[0]
    V = emb_weight.shape[0]
    D = W.shape[0]
    P = _project_table(emb_weight, W, b)

    info = plsc.get_sparse_core_info()
    NC, NS = info.num_cores, info.num_subcores
    NW = NC * NS
    assert B % NW == 0, (B, NW)
    b_per_w = B // NW

    C, NBUF = 0, 1
    for cand in range(128, 0, -8):
        if b_per_w % cand == 0:
            n_chunks = b_per_w // cand
            for nb in (5, 4, 6, 3, 2, 1):
                if n_chunks % nb == 0:
                    C, NBUF = cand, nb
                    break
            if C:
                break
    assert C > 0, b_per_w

    return _make_gather(B, V, D, NC, NS, b_per_w, C, NBUF)(P, t)


# speedup vs baseline: 9.3870x; 1.0191x over previous
import functools

import jax
import jax.numpy as jnp
from jax import lax
from jax.experimental import pallas as pl
from jax.experimental.pallas import tpu as pltpu
from jax.experimental.pallas import tpu_sc as plsc


def _project_body(emb_ref, w_ref, b_ref, out_ref):
    out_ref[...] = lax.dot_general(
        emb_ref[...], w_ref[...],
        (((1,), (1,)), ((), ())),
        preferred_element_type=jnp.float32,
    ) + b_ref[...]


def _project_table(emb_weight, W, b):
    m = emb_weight.shape[0]
    return pl.pallas_call(
        _project_body,
        out_shape=jax.ShapeDtypeStruct((m, W.shape[0]), jnp.float32),
    )(emb_weight, W, b.reshape(1, -1))


@functools.lru_cache(maxsize=None)
def _make_gather(B, V, D, NC, NS, b_per_w, C, NBUF):
    n_chunks = b_per_w // C
    n_groups = n_chunks // NBUF
    assert n_groups % 2 == 0
    n_pairs = n_groups // 2
    mesh = plsc.VectorSubcoreMesh(
        core_axis_name="c", subcore_axis_name="s",
        num_cores=NC, num_subcores=NS)

    @functools.partial(
        pl.kernel,
        out_type=jax.ShapeDtypeStruct((B, D), jnp.float32),
        mesh=mesh,
        scratch_types=[
            pltpu.VMEM_SHARED((V, D), jnp.float32),
            pltpu.VMEM((b_per_w,), jnp.int32),
            pltpu.VMEM((2, NBUF, C, D), jnp.float32),
        ] + [pltpu.SemaphoreType.DMA] * (4 * NBUF),
    )
    def gather_k(table_hbm, idx_hbm, out_hbm, shared_tab, idx_v, rows_v,
                 *sems):
        gsem = (sems[:NBUF], sems[NBUF:2 * NBUF])
        ssem = (sems[2 * NBUF:3 * NBUF], sems[3 * NBUF:])
        sid = lax.axis_index("s")
        wid = sid * NC + lax.axis_index("c")
        base = wid * b_per_w

        @pl.when(sid == 0)
        def _():
            pltpu.sync_copy(table_hbm, shared_tab)

        pltpu.sync_copy(idx_hbm.at[pl.ds(base, b_per_w)], idx_v)
        plsc.subcore_barrier()

        def fire_gather(grp, k, s):
            off = (grp * NBUF + k) * C
            pltpu.async_copy(
                shared_tab.at[idx_v.at[pl.ds(off, C)]],
                rows_v.at[s, k], gsem[s][k])

        def fire_store(grp, k, s):
            off = (grp * NBUF + k) * C
            pltpu.async_copy(
                rows_v.at[s, k], out_hbm.at[pl.ds(base + off, C)],
                ssem[s][k])

        def drain(sem, s, k):
            pltpu.make_async_copy(
                out_hbm.at[pl.ds(0, C)], rows_v.at[s, k], sem).wait()

        for k in range(NBUF):
            fire_gather(0, k, 0)

        def pair_body(g2, carry):
            a = 2 * g2
            for k in range(NBUF):
                @pl.when(g2 > 0)
                def _():
                    drain(ssem[1][k], 1, k)
                fire_gather(a + 1, k, 1)
            for k in range(NBUF):
                drain(gsem[0][k], 0, k)
                fire_store(a, k, 0)
            for k in range(NBUF):
                drain(ssem[0][k], 0, k)

                @pl.when(g2 < n_pairs - 1)
                def _():
                    fire_gather(a + 2, k, 0)
            for k in range(NBUF):
                drain(gsem[1][k], 1, k)
                fire_store(a + 1, k, 1)
            return carry

        lax.fori_loop(0, n_pairs, pair_body, 0)
        for k in range(NBUF):
            drain(ssem[1][k], 1, k)

    return gather_k


def kernel(t, emb_weight, W, b):
    B = t.shape[0]
    V = emb_weight.shape[0]
    D = W.shape[0]
    P = _project_table(emb_weight, W, b)

    info = plsc.get_sparse_core_info()
    NC, NS = info.num_cores, info.num_subcores
    NW = NC * NS
    assert B % NW == 0, (B, NW)
    b_per_w = B // NW

    C, NBUF = 0, 0
    for cand in range(128, 0, -8):
        if b_per_w % cand == 0:
            n_chunks = b_per_w // cand
            for nb in (5, 4, 6, 3, 2):
                if n_chunks % nb == 0 and (n_chunks // nb) % 2 == 0:
                    C, NBUF = cand, nb
                    break
            if C:
                break
    assert C > 0, b_per_w

    return _make_gather(B, V, D, NC, NS, b_per_w, C, NBUF)(P, t)
